# baseline (speedup 1.0000x reference)
import functools

import jax
import jax.numpy as jnp
from jax import lax
from jax.experimental import pallas as pl
from jax.experimental.pallas import tpu as pltpu
from jax.experimental.pallas import tpu_sc as plsc

_B, _L, _D = 4096, 200, 64
_NC, _NS = 2, 16
_NW = _NC * _NS
_RPW = _B // _NW
_POS = (10, 12, 13, 14, 43, 44)
_NEG = (11, 15, 45, 46)


def _sc_pool_dot(ids_flat, mask_flat, table, fcw):
  mesh = plsc.VectorSubcoreMesh(core_axis_name="c", subcore_axis_name="s")

  @functools.partial(
      pl.kernel,
      out_type=jax.ShapeDtypeStruct((_B, 16), jnp.float32),
      mesh=mesh,
      compiler_params=pltpu.CompilerParams(use_tc_tiling_on_sc=False),
      scratch_types=[
          pltpu.VMEM((_RPW * _L,), jnp.int32),
          pltpu.VMEM((_RPW * _L,), jnp.int32),
          pltpu.VMEM((_L, _D), jnp.float32),
          pltpu.VMEM((_L, _D), jnp.float32),
          pltpu.VMEM((_D,), jnp.float32),
          pltpu.VMEM((_RPW, 16), jnp.float32),
          pltpu.SemaphoreType.DMA,
          pltpu.SemaphoreType.DMA,
      ],
  )
  def k(ids_hbm, mask_hbm, table_hbm, fcw_hbm, out_hbm,
        ids_v, mask_v, buf0, buf1, fcw_v, out_v, sem0, sem1):
    cid = lax.axis_index("c")
    sid = lax.axis_index("s")
    wid = sid * _NC + cid
    base = pl.multiple_of(wid * (_RPW * _L), 8)

    pltpu.sync_copy(fcw_hbm, fcw_v)
    pltpu.sync_copy(ids_hbm.at[pl.ds(base, _RPW * _L)], ids_v)
    pltpu.sync_copy(mask_hbm.at[pl.ds(base, _RPW * _L)], mask_v)

    def mask_body(i, carry):
      o = pl.multiple_of(i * 128, 8)
      for u in range(8):
        sl = pl.ds(o + u * 16, 16)
        ids_v[sl] = ids_v[sl] * mask_v[sl]
      return carry
    lax.fori_loop(0, _RPW * _L // 128, mask_body, 0)

    fw = [fcw_v[pl.ds(c * 16, 16)] for c in range(4)]
    bufs = (buf0, buf1)
    sems = (sem0, sem1)

    def start_gather(r, buf, sem):
      off = pl.multiple_of(r * _L, 8)
      pltpu.async_copy(table_hbm.at[ids_v.at[pl.ds(off, 128)]],
                       buf.at[pl.ds(0, 128)], sem)
      off2 = pl.multiple_of(r * _L + 128, 8)
      pltpu.async_copy(table_hbm.at[ids_v.at[pl.ds(off2, _L - 128)]],
                       buf.at[pl.ds(128, _L - 128)], sem)

    def wait_gather(buf, sem):
      pltpu.make_async_copy(table_hbm.at[pl.ds(0, _L)], buf, sem).wait()

    def row_pool(r, buf):
      zero = jnp.zeros((16,), jnp.float32)

      def pb(t, a):
        a = list(a)
        o = t * 8
        for u in range(8):
          row = o + u
          for c in range(4):
            j = (u % 2) * 4 + c
            a[j] = a[j] + buf[row, pl.ds(c * 16, 16)]
        return tuple(a)

      accs = lax.fori_loop(0, _L // 8, pb, (zero,) * 8)
      s = ((accs[0] + accs[4]) * fw[0] + (accs[1] + accs[5]) * fw[1]
           + (accs[2] + accs[6]) * fw[2] + (accs[3] + accs[7]) * fw[3])
      out_v[r, :] = s

    start_gather(0, buf0, sem0)

    def outer(i, carry):
      for k2 in range(2):
        r = i * 2 + k2

        @pl.when(r + 1 < _RPW)
        def _():
          start_gather(r + 1, bufs[k2 ^ 1], sems[k2 ^ 1])

        wait_gather(bufs[k2], sems[k2])
        row_pool(r, bufs[k2])
      return carry
    lax.fori_loop(0, _RPW // 2, outer, 0)

    ob = pl.multiple_of(wid * _RPW, 8)
    pltpu.sync_copy(out_v, out_hbm.at[pl.ds(ob, _RPW), :])

  return k(ids_flat, mask_flat, table, fcw)


def _tc_combine(input_ids, attention_mask, dots, fc_b):
  def body(ids_ref, m_ref, d_ref, b_ref, o_ref):
    ids = ids_ref[...]
    m = m_ref[...].astype(jnp.float32)
    cnt = jnp.maximum(jnp.sum(m, axis=1), 1e-8)
    bonus = jnp.zeros((_B,), jnp.float32)
    for c in _POS:
      bonus = bonus + 0.5 * jnp.any(ids == c, axis=1).astype(jnp.float32)
    for c in _NEG:
      bonus = bonus - 0.5 * jnp.any(ids == c, axis=1).astype(jnp.float32)
    dot = jnp.sum(d_ref[...], axis=1)
    o_ref[...] = dot / cnt + b_ref[0] + bonus

  return pl.pallas_call(
      body,
      in_specs=[
          pl.BlockSpec(memory_space=pltpu.VMEM),
          pl.BlockSpec(memory_space=pltpu.VMEM),
          pl.BlockSpec(memory_space=pltpu.VMEM),
          pl.BlockSpec(memory_space=pltpu.SMEM),
      ],
      out_specs=pl.BlockSpec(memory_space=pltpu.VMEM),
      out_shape=jax.ShapeDtypeStruct((_B,), jnp.float32),
  )(input_ids, attention_mask, dots, fc_b)


@jax.jit
def kernel(input_ids, attention_mask, table, fc_w, fc_b):
  ids_flat = input_ids.reshape(-1)
  mask_flat = attention_mask.reshape(-1)
  fcw = fc_w.reshape(-1).astype(jnp.float32)
  dots = _sc_pool_dot(ids_flat, mask_flat, table, fcw)
  return _tc_combine(input_ids, attention_mask, dots, fc_b)


# speedup vs baseline: 13.5486x; 13.5486x over previous
import functools

import jax
import jax.numpy as jnp
from jax import lax
from jax.experimental import pallas as pl
from jax.experimental.pallas import tpu as pltpu
from jax.experimental.pallas import tpu_sc as plsc

_B, _L, _D = 4096, 200, 64
_V = 1000000
_NC, _NS = 2, 16
_NW = _NC * _NS
_TPW = _B * _L // _NW
_POS = (10, 12, 13, 14, 43, 44)
_NEG = (11, 15, 45, 46)
_PBLK = 8000


def _tc_project(table, fcw):
  grid = _V // _PBLK

  def body(w_ref, t_ref, o_ref):
    s = jnp.sum(t_ref[...] * w_ref[...], axis=1)
    o_ref[...] = s.reshape(1, 8, _PBLK // 8)

  return pl.pallas_call(
      body,
      grid=(grid,),
      in_specs=[
          pl.BlockSpec((1, _D), lambda i: (0, 0)),
          pl.BlockSpec((_PBLK, _D), lambda i: (i, 0)),
      ],
      out_specs=pl.BlockSpec((1, 8, _PBLK // 8), lambda i: (i, 0, 0)),
      out_shape=jax.ShapeDtypeStruct((grid, 8, _PBLK // 8), jnp.float32),
  )(fcw.reshape(1, _D), table)


def _sc_gather(ids_flat, p):
  mesh = plsc.VectorSubcoreMesh(core_axis_name="c", subcore_axis_name="s")

  @functools.partial(
      pl.kernel,
      out_type=jax.ShapeDtypeStruct((_B * _L,), jnp.float32),
      mesh=mesh,
      compiler_params=pltpu.CompilerParams(use_tc_tiling_on_sc=False),
      scratch_types=[
          pltpu.VMEM_SHARED((_V,), jnp.float32),
          pltpu.VMEM((_TPW,), jnp.int32),
          pltpu.VMEM((_TPW,), jnp.float32),
          pltpu.SemaphoreType.DMA,
      ],
  )
  def g(ids_hbm, p_hbm, out_hbm, p_s, ids_v, vals_v, sem):
    cid = lax.axis_index("c")
    sid = lax.axis_index("s")
    wid = sid * _NC + cid
    tokbase = pl.multiple_of(wid * _TPW, 8)

    pltpu.sync_copy(ids_hbm.at[pl.ds(tokbase, _TPW)], ids_v)

    @pl.when(sid == 0)
    def _():
      pltpu.sync_copy(p_hbm, p_s)

    plsc.subcore_barrier()

    def g_body(j, carry):
      o = pl.multiple_of(j * 128, 8)
      pltpu.async_copy(p_s.at[ids_v.at[pl.ds(o, 128)]],
                       vals_v.at[pl.ds(o, 128)], sem)
      return carry
    lax.fori_loop(0, _TPW // 128, g_body, 0)

    pltpu.make_async_copy(p_hbm.at[pl.ds(0, _TPW)], vals_v, sem).wait()

    pltpu.sync_copy(vals_v, out_hbm.at[pl.ds(tokbase, _TPW)])

  return g(ids_flat, p)


def _tc_combine(input_ids, attention_mask, vals, fc_b):
  def body(ids_ref, m_ref, v_ref, b_ref, o_ref):
    ids = ids_ref[...]
    m = m_ref[...].astype(jnp.float32)
    cnt = jnp.maximum(jnp.sum(m, axis=1), 1e-8)
    num = jnp.sum(v_ref[...] * m, axis=1)
    bonus = jnp.zeros((_B,), jnp.float32)
    for c in _POS:
      bonus = bonus + 0.5 * jnp.any(ids == c, axis=1).astype(jnp.float32)
    for c in _NEG:
      bonus = bonus - 0.5 * jnp.any(ids == c, axis=1).astype(jnp.float32)
    o_ref[...] = num / cnt + b_ref[0] + bonus

  return pl.pallas_call(
      body,
      in_specs=[
          pl.BlockSpec(memory_space=pltpu.VMEM),
          pl.BlockSpec(memory_space=pltpu.VMEM),
          pl.BlockSpec(memory_space=pltpu.VMEM),
          pl.BlockSpec(memory_space=pltpu.SMEM),
      ],
      out_specs=pl.BlockSpec(memory_space=pltpu.VMEM),
      out_shape=jax.ShapeDtypeStruct((_B,), jnp.float32),
  )(input_ids, attention_mask, vals, fc_b)


@jax.jit
def kernel(input_ids, attention_mask, table, fc_w, fc_b):
  fcw = fc_w.reshape(-1).astype(jnp.float32)
  p = _tc_project(table, fcw).reshape(_V)
  vals = _sc_gather(input_ids.reshape(-1), p)
  return _tc_combine(input_ids, attention_mask, vals.reshape(_B, _L), fc_b)
